# Initial kernel scaffold; baseline (speedup 1.0000x reference)
import jax
import jax.numpy as jnp
from jax.experimental import pallas as pl


def kernel(next_logits, bias_row, delta):
    raise NotImplementedError("write your pallas kernel here")


# speedup vs baseline: 79.9213x; 79.9213x over previous
import jax
import jax.numpy as jnp
from jax.experimental import pallas as pl

_TOP_P = 0.98
_ROWS = 8


def _monokey(i):
    return i ^ ((i >> 31) & jnp.int32(0x7FFFFFFF))


def _favg(lo, hi):
    return (lo >> 1) + (hi >> 1) + (lo & hi & jnp.int32(1))


def _topp_body(nl_ref, br_ref, d_ref, out_ref):
    nl = nl_ref[...]
    br = br_ref[...]
    dl = d_ref[...]
    R, V = nl.shape

    ln = jnp.sqrt(jnp.sum(nl * nl, axis=-1, keepdims=True))
    bn = jnp.sqrt(jnp.sum(br * br, axis=-1, keepdims=True))
    scale = jnp.where(bn > 1e-12, ln / jnp.maximum(bn, 1e-12), 1.0)
    biased = nl + scale * dl

    m = jnp.max(biased, axis=-1, keepdims=True)
    p = jnp.exp(biased - m)
    Z = jnp.sum(p, axis=-1, keepdims=True)
    P = _TOP_P * Z

    rmin = jnp.min(biased, axis=-1, keepdims=True)
    lo0 = _monokey(jax.lax.bitcast_convert_type(rmin, jnp.int32)) - 1
    hi0 = _monokey(jax.lax.bitcast_convert_type(m, jnp.int32))

    def vbody(_, carry):
        lo, hi, f_hi = carry
        mid = _favg(lo, hi)
        t = jax.lax.bitcast_convert_type(_monokey(mid), jnp.float32)
        f = jnp.sum(jnp.where(biased > t, p, 0.0), axis=-1, keepdims=True)
        gt = f > P
        return (jnp.where(gt, mid, lo),
                jnp.where(gt, hi, mid),
                jnp.where(gt, f_hi, f))

    _, hi, s_strict = jax.lax.fori_loop(
        0, 32, vbody, (lo0, hi0, jnp.zeros_like(Z)))
    vstar = jax.lax.bitcast_convert_type(_monokey(hi), jnp.float32)

    eq = biased == vstar
    n_ties = jnp.sum(eq.astype(jnp.int32), axis=-1, keepdims=True)
    p_v = jnp.exp(vstar - m)
    r_f = jnp.floor((P - s_strict) / p_v) + 1.0
    r_f = jnp.minimum(r_f, n_ties.astype(jnp.float32))
    r = jnp.maximum(r_f.astype(jnp.int32), 1)

    col = jax.lax.broadcasted_iota(jnp.int32, (R, V), 1)

    def ibody(_, carry):
        ilo, ihi = carry
        imid = _favg(ilo, ihi)
        cnt = jnp.sum(jnp.where(eq & (col <= imid), 1, 0),
                      axis=-1, keepdims=True)
        less = cnt < r
        return (jnp.where(less, imid, ilo), jnp.where(less, ihi, imid))

    ilo0 = jnp.full_like(n_ties, -1)
    ihi0 = jnp.full_like(n_ties, V - 1)
    _, cstar = jax.lax.fori_loop(0, 17, ibody, (ilo0, ihi0))

    keep = (biased > vstar) | (eq & (col <= cstar))
    out_ref[...] = jnp.where(keep, biased, -jnp.inf)


def kernel(next_logits, bias_row, delta):
    B, V = next_logits.shape
    spec = pl.BlockSpec((_ROWS, V), lambda i: (i, 0))
    return pl.pallas_call(
        _topp_body,
        grid=(B // _ROWS,),
        in_specs=[spec, spec, spec],
        out_specs=spec,
        out_shape=jax.ShapeDtypeStruct((B, V), jnp.float32),
    )(next_logits, bias_row, delta)
